# Initial kernel scaffold; baseline (speedup 1.0000x reference)
import jax
import jax.numpy as jnp
from jax.experimental import pallas as pl


def kernel(x, edge_index, xe, W_l0, b_l0, W_r0, W_l1, b_l1, W_r1, W_proj, b_proj):
    raise NotImplementedError("write your pallas kernel here")


# speedup vs baseline: 2.1441x; 2.1441x over previous
import functools

import jax
import jax.numpy as jnp
from jax import lax
from jax.experimental import pallas as pl
from jax.experimental.pallas import tpu as pltpu
from jax.experimental.pallas import tpu_sc as plsc

N = 10000
E = 320000
D = 128
C = 64

NC = 2
NS = 16
NW = NC * NS
EP = E // NW
K = 80
NCH = EP // K
RP = 1000
NRT = N // RP
RZ = 40
LG = D // 16


def _make_edge_agg(want_cnt):
    mesh = plsc.VectorSubcoreMesh(core_axis_name="c", subcore_axis_name="s")
    part_ty = jax.ShapeDtypeStruct((NC, N, D), jnp.float32)
    cnt_ty = jax.ShapeDtypeStruct((N,), jnp.float32)
    out_type = [part_ty, cnt_ty, cnt_ty] if want_cnt else part_ty
    scratch = [
        pltpu.VMEM_SHARED((N, D), jnp.float32),
        pltpu.VMEM((K,), jnp.int32),
        pltpu.VMEM((K,), jnp.int32),
        pltpu.VMEM((K, D), jnp.float32),
        pltpu.VMEM((K, D), jnp.float32),
        pltpu.VMEM((RZ, D), jnp.float32),
        pltpu.SemaphoreType.DMA,
    ]
    if want_cnt:
        scratch += [
            pltpu.VMEM_SHARED((N,), jnp.float32),
            pltpu.VMEM((K,), jnp.float32),
            pltpu.VMEM((1024,), jnp.float32),
        ]

    def body(x_hbm, src_hbm, dst_hbm, xe_hbm, part_out, *rest):
        if want_cnt:
            (cnt_out0, cnt_out1, acc_sh, idx_s, idx_d, xrows, xe_v, zbuf,
             sem, cnt_sh, ones_v, zcnt) = rest
        else:
            acc_sh, idx_s, idx_d, xrows, xe_v, zbuf, sem = rest
        c = lax.axis_index("c")
        s = lax.axis_index("s")
        wid = s * NC + c
        zero16 = jnp.zeros((16,), jnp.float32)

        def zrow(r, carry):
            for g in range(LG):
                zbuf[r, pl.ds(g * 16, 16)] = zero16
            return carry
        lax.fori_loop(0, RZ, zrow, 0)

        @pl.when(s < NRT)
        def _():
            def zacc(j, carry):
                pltpu.sync_copy(zbuf, acc_sh.at[pl.ds(s * RP + j * RZ, RZ)])
                return carry
            lax.fori_loop(0, RP // RZ, zacc, 0)

        if want_cnt:
            def zc(i, carry):
                zcnt[pl.ds(i * 16, 16)] = zero16
                return carry
            lax.fori_loop(0, 1024 // 16, zc, 0)

            def fones(i, carry):
                ones_v[pl.ds(i * 16, 16)] = zero16 + 1.0
                return carry
            lax.fori_loop(0, K // 16, fones, 0)

            @pl.when(s < NRT)
            def _():
                pltpu.sync_copy(zcnt.at[pl.ds(0, RP)],
                                cnt_sh.at[pl.ds(s * RP, RP)])

        plsc.subcore_barrier()

        ebase = wid * EP

        def chunk(j, carry):
            base = ebase + j * K
            pltpu.sync_copy(src_hbm.at[pl.ds(base, K)], idx_s)
            pltpu.sync_copy(dst_hbm.at[pl.ds(base, K)], idx_d)
            pltpu.sync_copy(xe_hbm.at[pl.ds(base, K)], xe_v)
            pltpu.async_copy(x_hbm.at[idx_s], xrows, sem).wait()

            def row(r, rc):
                for g in range(LG):
                    sl = pl.ds(g * 16, 16)
                    xrows[r, sl] = jnp.maximum(xrows[r, sl] + xe_v[r, sl], 0.0)
                return rc
            lax.fori_loop(0, K, row, 0, unroll=2)

            pltpu.sync_copy(xrows, acc_sh.at[idx_d], add=True)
            if want_cnt:
                pltpu.sync_copy(ones_v, cnt_sh.at[idx_d], add=True)
            return carry
        lax.fori_loop(0, NCH, chunk, 0)

        plsc.subcore_barrier()

        @pl.when(s < NRT)
        def _():
            pltpu.sync_copy(acc_sh.at[pl.ds(s * RP, RP)],
                            part_out.at[c, pl.ds(s * RP, RP)])
        if want_cnt:
            @pl.when(s < NRT)
            def _():
                pltpu.sync_copy(cnt_sh.at[pl.ds(s * RP, RP)],
                                zcnt.at[pl.ds(0, RP)])

            @pl.when((s < NRT) & (c == 0))
            def _():
                pltpu.sync_copy(zcnt.at[pl.ds(0, RP)],
                                cnt_out0.at[pl.ds(s * RP, RP)])

            @pl.when((s < NRT) & (c == 1))
            def _():
                pltpu.sync_copy(zcnt.at[pl.ds(0, RP)],
                                cnt_out1.at[pl.ds(s * RP, RP)])

    return pl.kernel(body, out_type=out_type, mesh=mesh,
                     scratch_types=scratch)


_edge_agg_cnt = _make_edge_agg(True)
_edge_agg = _make_edge_agg(False)

R = 1000


def _tc0_body(part, cnt0, cnt1, x, wl, bl, wr, out):
    summ = part[0] + part[1]
    deg = cnt0[...] + cnt1[...]
    agg = summ / jnp.maximum(deg, 1.0)
    h = (jnp.dot(agg, wl[...], preferred_element_type=jnp.float32) + bl[...]
         + jnp.dot(x[...], wr[...], preferred_element_type=jnp.float32))
    out[...] = jnp.maximum(h, 0.0)


_tc0 = pl.pallas_call(
    _tc0_body,
    grid=(N // R,),
    in_specs=[
        pl.BlockSpec((NC, R, D), lambda i: (0, i, 0)),
        pl.BlockSpec((R, 1), lambda i: (i, 0)),
        pl.BlockSpec((R, 1), lambda i: (i, 0)),
        pl.BlockSpec((R, D), lambda i: (i, 0)),
        pl.BlockSpec((D, D), lambda i: (0, 0)),
        pl.BlockSpec((1, D), lambda i: (0, 0)),
        pl.BlockSpec((D, D), lambda i: (0, 0)),
    ],
    out_specs=pl.BlockSpec((R, D), lambda i: (i, 0)),
    out_shape=jax.ShapeDtypeStruct((N, D), jnp.float32),
)


def _tc1_body(part, cnt0, cnt1, h, wl, bl, wr, wp, bp, out, g):
    i = pl.program_id(0)
    summ = part[0] + part[1]
    deg = cnt0[...] + cnt1[...]
    agg = summ / jnp.maximum(deg, 1.0)
    h2 = (jnp.dot(agg, wl[...], preferred_element_type=jnp.float32) + bl[...]
          + jnp.dot(h[...], wr[...], preferred_element_type=jnp.float32))

    @pl.when(i == 0)
    def _():
        g[...] = jnp.zeros_like(g)
    g[...] += jnp.sum(h2, axis=0, keepdims=True) * (1.0 / N)
    out[...] = (jnp.dot(jnp.maximum(h2, 0.0), wp[...],
                        preferred_element_type=jnp.float32) + bp[...])


_tc1 = pl.pallas_call(
    _tc1_body,
    grid=(N // R,),
    in_specs=[
        pl.BlockSpec((NC, R, D), lambda i: (0, i, 0)),
        pl.BlockSpec((R, 1), lambda i: (i, 0)),
        pl.BlockSpec((R, 1), lambda i: (i, 0)),
        pl.BlockSpec((R, D), lambda i: (i, 0)),
        pl.BlockSpec((D, D), lambda i: (0, 0)),
        pl.BlockSpec((1, D), lambda i: (0, 0)),
        pl.BlockSpec((D, D), lambda i: (0, 0)),
        pl.BlockSpec((D, C), lambda i: (0, 0)),
        pl.BlockSpec((1, C), lambda i: (0, 0)),
    ],
    out_specs=[
        pl.BlockSpec((R, C), lambda i: (i, 0)),
        pl.BlockSpec((1, D), lambda i: (0, 0)),
    ],
    out_shape=[
        jax.ShapeDtypeStruct((N, C), jnp.float32),
        jax.ShapeDtypeStruct((1, D), jnp.float32),
    ],
)


def kernel(x, edge_index, xe, W_l0, b_l0, W_r0, W_l1, b_l1, W_r1, W_proj,
           b_proj):
    src = edge_index[0]
    dst = edge_index[1]
    part0, cnt0, cnt1 = _edge_agg_cnt(x, src, dst, xe)
    cnt0 = cnt0.reshape(N, 1)
    cnt1 = cnt1.reshape(N, 1)
    h1 = _tc0(part0, cnt0, cnt1, x, W_l0, b_l0.reshape(1, D), W_r0)
    part1 = _edge_agg(h1, src, dst, xe)
    h_out, g = _tc1(part1, cnt0, cnt1, h1, W_l1, b_l1.reshape(1, D), W_r1,
                    W_proj, b_proj.reshape(1, C))
    return (h_out, g)


# speedup vs baseline: 3.0202x; 1.4086x over previous
import functools

import jax
import jax.numpy as jnp
from jax import lax
from jax.experimental import pallas as pl
from jax.experimental.pallas import tpu as pltpu
from jax.experimental.pallas import tpu_sc as plsc

N = 10000
E = 320000
D = 128
C = 64

NC = 2
NS = 16
NW = NC * NS
EP = E // NW
K = 40
G = 4
GK = G * K
NG = EP // GK
TAIL = (EP - NG * GK) // K
RP = 1000
NRT = N // RP
RZ = 8
LG = D // 16


def _make_edge_agg(want_cnt):
    mesh = plsc.VectorSubcoreMesh(core_axis_name="c", subcore_axis_name="s")
    xrow_rows = N + 8 * NRT if want_cnt else N
    part_ty = jax.ShapeDtypeStruct((NC, xrow_rows, D), jnp.float32)
    scratch = [
        pltpu.VMEM_SHARED((N, D), jnp.float32),
        pltpu.VMEM((2 * G, K), jnp.int32),
        pltpu.VMEM((2 * G, K), jnp.int32),
        pltpu.VMEM((G, K, D), jnp.float32),
        pltpu.VMEM((GK, D), jnp.float32),
        pltpu.VMEM((RZ, D), jnp.float32),
        pltpu.SemaphoreType.DMA((2 * G,)),
        pltpu.SemaphoreType.DMA((2 * G,)),
        pltpu.SemaphoreType.DMA((G,)),
        pltpu.SemaphoreType.DMA,
        pltpu.SemaphoreType.DMA((G,)),
    ]
    if want_cnt:
        scratch += [
            pltpu.VMEM_SHARED((N,), jnp.float32),
            pltpu.VMEM((48,), jnp.float32),
            pltpu.VMEM((1024,), jnp.float32),
            pltpu.SemaphoreType.DMA((G,)),
        ]

    def body(x_hbm, src_hbm, dst_hbm, xe_hbm, part_out, *rest):
        if want_cnt:
            (acc_sh, idx_s, idx_d, xrows, xe_v, zbuf,
             isem, dsem, gsem, xesem, ssem, cnt_sh, ones_v, zcnt,
             csem) = rest
        else:
            (acc_sh, idx_s, idx_d, xrows, xe_v, zbuf,
             isem, dsem, gsem, xesem, ssem) = rest
        c = lax.axis_index("c")
        s = lax.axis_index("s")
        wid = s * NC + c
        zero16 = jnp.zeros((16,), jnp.float32)

        def zrow(r, carry):
            for g in range(LG):
                zbuf[r, pl.ds(g * 16, 16)] = zero16
            return carry
        lax.fori_loop(0, RZ, zrow, 0)

        @pl.when(s < NRT)
        def _():
            def zacc(j, carry):
                pltpu.sync_copy(zbuf, acc_sh.at[pl.ds(s * RP + j * RZ, RZ)])
                return carry
            lax.fori_loop(0, RP // RZ, zacc, 0)

        if want_cnt:
            def zc(i, carry):
                zcnt[pl.ds(i * 16, 16)] = zero16
                return carry
            lax.fori_loop(0, 1024 // 16, zc, 0)

            def fones(i, carry):
                ones_v[pl.ds(i * 16, 16)] = zero16 + 1.0
                return carry
            lax.fori_loop(0, 48 // 16, fones, 0)

            @pl.when(s < NRT)
            def _():
                pltpu.sync_copy(zcnt.at[pl.ds(0, RP)],
                                cnt_sh.at[pl.ds(s * RP, RP)])

        plsc.subcore_barrier()

        ebase = wid * EP

        def issue_idx_group(t, p, ng):
            for q in range(ng):
                base = ebase + t * GK + q * K
                row = G * p + q
                pltpu.async_copy(src_hbm.at[pl.ds(base, K)],
                                 idx_s.at[row], isem.at[row])
                pltpu.async_copy(dst_hbm.at[pl.ds(base, K)],
                                 idx_d.at[row], dsem.at[row])

        def wait_idx(row):
            pltpu.make_async_copy(src_hbm.at[pl.ds(0, K)], idx_s.at[row],
                                  isem.at[row]).wait()
            pltpu.make_async_copy(dst_hbm.at[pl.ds(0, K)], idx_d.at[row],
                                  dsem.at[row]).wait()

        def compute(q):
            def row(r, rc):
                for g in range(LG):
                    sl = pl.ds(g * 16, 16)
                    xrows[q, r, sl] = jnp.maximum(
                        xrows[q, r, sl] + xe_v[q * K + r, sl], 0.0)
                return rc
            lax.fori_loop(0, K, row, 0, unroll=2)

        def run_group(t, p, ng, prefetch):
            xe_d = pltpu.make_async_copy(
                xe_hbm.at[pl.ds(ebase + t * GK, ng * K)],
                xe_v.at[pl.ds(0, ng * K)], xesem)
            xe_d.start()
            gds = []
            for q in range(ng):
                row = G * p + q
                wait_idx(row)
                gd = pltpu.make_async_copy(x_hbm.at[idx_s.at[row]],
                                           xrows.at[q], gsem.at[q])
                gd.start()
                gds.append(gd)
            if prefetch:
                @pl.when(t + 1 < NG)
                def _():
                    issue_idx_group(t + 1, 1 - p, G)

                @pl.when(t + 1 == NG)
                def _():
                    issue_idx_group(t + 1, 1 - p, TAIL)
            xe_d.wait()
            sds = []
            for q in range(ng):
                row = G * p + q
                gds[q].wait()
                compute(q)
                sd = pltpu.make_async_copy(xrows.at[q],
                                           acc_sh.at[idx_d.at[row]],
                                           ssem.at[q])
                sd.start(add=True)
                sds.append(sd)
                if want_cnt:
                    cd = pltpu.make_async_copy(ones_v.at[pl.ds(0, K)],
                                               cnt_sh.at[idx_d.at[row]],
                                               csem.at[q])
                    cd.start(add=True)
                    sds.append(cd)
            for sd in sds:
                sd.wait()

        issue_idx_group(0, 0, G)

        def group_body(t, carry):
            @pl.when((t & 1) == 0)
            def _():
                run_group(t, 0, G, True)

            @pl.when((t & 1) == 1)
            def _():
                run_group(t, 1, G, True)
            return carry
        lax.fori_loop(0, NG, group_body, 0)
        run_group(NG, NG % 2, TAIL, False)

        plsc.subcore_barrier()

        @pl.when(s < NRT)
        def _():
            pltpu.sync_copy(acc_sh.at[pl.ds(s * RP, RP)],
                            part_out.at[c, pl.ds(s * RP, RP)])
        if want_cnt:
            @pl.when(s < NRT)
            def _():
                pltpu.sync_copy(cnt_sh.at[pl.ds(s * RP, RP)],
                                zcnt.at[pl.ds(0, RP)])
                for r in range(8):
                    for g in range(LG):
                        zbuf[r, pl.ds(g * 16, 16)] = (
                            zcnt[pl.ds(r * 128 + g * 16, 16)])
                pltpu.sync_copy(zbuf, part_out.at[c, pl.ds(N + s * 8, 8)])

    return pl.kernel(body, out_type=part_ty, mesh=mesh,
                     scratch_types=scratch)


_edge_agg_cnt = _make_edge_agg(True)
_edge_agg = _make_edge_agg(False)

R = 1000


def _tc0_body(part, cnt0, cnt1, x, wl, bl, wr, out):
    summ = part[0] + part[1]
    deg = cnt0[...] + cnt1[...]
    agg = summ / jnp.maximum(deg, 1.0)
    h = (jnp.dot(agg, wl[...], preferred_element_type=jnp.float32) + bl[...]
         + jnp.dot(x[...], wr[...], preferred_element_type=jnp.float32))
    out[...] = jnp.maximum(h, 0.0)


_tc0 = pl.pallas_call(
    _tc0_body,
    grid=(N // R,),
    in_specs=[
        pl.BlockSpec((NC, R, D), lambda i: (0, i, 0)),
        pl.BlockSpec((R, 1), lambda i: (i, 0)),
        pl.BlockSpec((R, 1), lambda i: (i, 0)),
        pl.BlockSpec((R, D), lambda i: (i, 0)),
        pl.BlockSpec((D, D), lambda i: (0, 0)),
        pl.BlockSpec((1, D), lambda i: (0, 0)),
        pl.BlockSpec((D, D), lambda i: (0, 0)),
    ],
    out_specs=pl.BlockSpec((R, D), lambda i: (i, 0)),
    out_shape=jax.ShapeDtypeStruct((N, D), jnp.float32),
)


def _tc1_body(part, cnt0, cnt1, h, wl, bl, wr, wp, bp, out, g):
    i = pl.program_id(0)
    summ = part[0] + part[1]
    deg = cnt0[...] + cnt1[...]
    agg = summ / jnp.maximum(deg, 1.0)
    h2 = (jnp.dot(agg, wl[...], preferred_element_type=jnp.float32) + bl[...]
          + jnp.dot(h[...], wr[...], preferred_element_type=jnp.float32))

    @pl.when(i == 0)
    def _():
        g[...] = jnp.zeros_like(g)
    g[...] += jnp.sum(h2, axis=0, keepdims=True) * (1.0 / N)
    out[...] = (jnp.dot(jnp.maximum(h2, 0.0), wp[...],
                        preferred_element_type=jnp.float32) + bp[...])


_tc1 = pl.pallas_call(
    _tc1_body,
    grid=(N // R,),
    in_specs=[
        pl.BlockSpec((NC, R, D), lambda i: (0, i, 0)),
        pl.BlockSpec((R, 1), lambda i: (i, 0)),
        pl.BlockSpec((R, 1), lambda i: (i, 0)),
        pl.BlockSpec((R, D), lambda i: (i, 0)),
        pl.BlockSpec((D, D), lambda i: (0, 0)),
        pl.BlockSpec((1, D), lambda i: (0, 0)),
        pl.BlockSpec((D, D), lambda i: (0, 0)),
        pl.BlockSpec((D, C), lambda i: (0, 0)),
        pl.BlockSpec((1, C), lambda i: (0, 0)),
    ],
    out_specs=[
        pl.BlockSpec((R, C), lambda i: (i, 0)),
        pl.BlockSpec((1, D), lambda i: (0, 0)),
    ],
    out_shape=[
        jax.ShapeDtypeStruct((N, C), jnp.float32),
        jax.ShapeDtypeStruct((1, D), jnp.float32),
    ],
)


def kernel(x, edge_index, xe, W_l0, b_l0, W_r0, W_l1, b_l1, W_r1, W_proj,
           b_proj):
    src = edge_index[0]
    dst = edge_index[1]
    part0 = _edge_agg_cnt(x, src, dst, xe)
    cnt0 = part0[0, N:].reshape(NRT, 1024)[:, :RP].reshape(N, 1)
    cnt1 = part0[1, N:].reshape(NRT, 1024)[:, :RP].reshape(N, 1)
    h1 = _tc0(part0, cnt0, cnt1, x, W_l0, b_l0.reshape(1, D), W_r0)
    part1 = _edge_agg(h1, src, dst, xe)
    h_out, g = _tc1(part1, cnt0, cnt1, h1, W_l1, b_l1.reshape(1, D), W_r1,
                    W_proj, b_proj.reshape(1, C))
    return (h_out, g)


# speedup vs baseline: 3.9700x; 1.3145x over previous
import functools

import jax
import jax.numpy as jnp
from jax import lax
from jax.experimental import pallas as pl
from jax.experimental.pallas import tpu as pltpu
from jax.experimental.pallas import tpu_sc as plsc

N = 10000
E = 320000
D = 128
C = 64

NC = 2
NS = 16
NW = NC * NS
EP = E // NW
K = 40
NCH = EP // K
NQ = 4
RP = 1000
NRT = N // RP
RZ = 8
LG = D // 16


def _make_edge_agg(want_cnt):
    mesh = plsc.VectorSubcoreMesh(core_axis_name="c", subcore_axis_name="s")
    xrow_rows = N + 8 * NRT if want_cnt else N
    part_ty = jax.ShapeDtypeStruct((NC, xrow_rows, D), jnp.float32)
    scratch = [
        pltpu.VMEM_SHARED((N, D), jnp.float32),
        pltpu.VMEM((NQ, K), jnp.int32),
        pltpu.VMEM((NQ, K), jnp.int32),
        pltpu.VMEM((NQ, K, D), jnp.float32),
        pltpu.VMEM((NQ, K, D), jnp.float32),
        pltpu.VMEM((RZ, D), jnp.float32),
        pltpu.SemaphoreType.DMA((NQ,)),
        pltpu.SemaphoreType.DMA((NQ,)),
        pltpu.SemaphoreType.DMA((NQ,)),
        pltpu.SemaphoreType.DMA((NQ,)),
        pltpu.SemaphoreType.DMA((NQ,)),
    ]
    if want_cnt:
        scratch += [
            pltpu.VMEM_SHARED((N,), jnp.float32),
            pltpu.VMEM((48,), jnp.float32),
            pltpu.VMEM((1024,), jnp.float32),
            pltpu.SemaphoreType.DMA((NQ,)),
        ]

    def body(x_hbm, src_hbm, dst_hbm, xe_hbm, part_out, *rest):
        if want_cnt:
            (acc_sh, idx_s, idx_d, xrows, xe_v, zbuf,
             isem, dsem, gsem, xesem, ssem, cnt_sh, ones_v, zcnt,
             csem) = rest
        else:
            (acc_sh, idx_s, idx_d, xrows, xe_v, zbuf,
             isem, dsem, gsem, xesem, ssem) = rest
        c = lax.axis_index("c")
        s = lax.axis_index("s")
        wid = s * NC + c
        zero16 = jnp.zeros((16,), jnp.float32)

        def zrow(r, carry):
            for g in range(LG):
                zbuf[r, pl.ds(g * 16, 16)] = zero16
            return carry
        lax.fori_loop(0, RZ, zrow, 0)

        @pl.when(s < NRT)
        def _():
            def zacc(j, carry):
                pltpu.sync_copy(zbuf, acc_sh.at[pl.ds(s * RP + j * RZ, RZ)])
                return carry
            lax.fori_loop(0, RP // RZ, zacc, 0)

        if want_cnt:
            def zc(i, carry):
                zcnt[pl.ds(i * 16, 16)] = zero16
                return carry
            lax.fori_loop(0, 1024 // 16, zc, 0)

            def fones(i, carry):
                ones_v[pl.ds(i * 16, 16)] = zero16 + 1.0
                return carry
            lax.fori_loop(0, 48 // 16, fones, 0)

            @pl.when(s < NRT)
            def _():
                pltpu.sync_copy(zcnt.at[pl.ds(0, RP)],
                                cnt_sh.at[pl.ds(s * RP, RP)])

        plsc.subcore_barrier()

        ebase = wid * EP

        def issue_idx(j, q):
            base = ebase + j * K
            pltpu.async_copy(src_hbm.at[pl.ds(base, K)], idx_s.at[q],
                             isem.at[q])
            pltpu.async_copy(dst_hbm.at[pl.ds(base, K)], idx_d.at[q],
                             dsem.at[q])

        def issue_xe(j, q):
            pltpu.async_copy(xe_hbm.at[pl.ds(ebase + j * K, K)],
                             xe_v.at[q], xesem.at[q])

        def wait_xe(q):
            pltpu.make_async_copy(xe_hbm.at[pl.ds(0, K)], xe_v.at[q],
                                  xesem.at[q]).wait()

        def issue_gather(q):
            pltpu.make_async_copy(src_hbm.at[pl.ds(0, K)], idx_s.at[q],
                                  isem.at[q]).wait()
            pltpu.async_copy(x_hbm.at[idx_s.at[q]], xrows.at[q],
                             gsem.at[q])

        def wait_gather(q):
            pltpu.make_async_copy(x_hbm.at[idx_s.at[q]], xrows.at[q],
                                  gsem.at[q]).wait()

        def issue_scatter(q):
            pltpu.make_async_copy(dst_hbm.at[pl.ds(0, K)], idx_d.at[q],
                                  dsem.at[q]).wait()
            pltpu.async_copy(xrows.at[q], acc_sh.at[idx_d.at[q]],
                             ssem.at[q], add=True)
            if want_cnt:
                pltpu.async_copy(ones_v.at[pl.ds(0, K)],
                                 cnt_sh.at[idx_d.at[q]],
                                 csem.at[q], add=True)

        def wait_scatter(q):
            pltpu.make_async_copy(xrows.at[q], acc_sh.at[idx_d.at[q]],
                                  ssem.at[q]).wait()
            if want_cnt:
                pltpu.make_async_copy(ones_v.at[pl.ds(0, K)],
                                      cnt_sh.at[idx_d.at[q]],
                                      csem.at[q]).wait()

        def compute(q):
            def row(r, rc):
                for g in range(LG):
                    sl = pl.ds(g * 16, 16)
                    xrows[q, r, sl] = jnp.maximum(
                        xrows[q, r, sl] + xe_v[q, r, sl], 0.0)
                return rc
            lax.fori_loop(0, K, row, 0, unroll=2)

        issue_idx(0, 0)
        issue_idx(1, 1)
        issue_xe(0, 0)
        issue_gather(0)

        def chunk(j, carry):
            u = j & 3
            for q in range(NQ):
                @pl.when(u == q)
                def _(q=q):
                    @pl.when(j >= 2)
                    def _():
                        wait_scatter((q + 2) & 3)

                    @pl.when(j + 2 < NCH)
                    def _():
                        issue_idx(j + 2, (q + 2) & 3)

                    @pl.when(j + 1 < NCH)
                    def _():
                        issue_xe(j + 1, (q + 1) & 3)
                        issue_gather((q + 1) & 3)

                    wait_gather(q)
                    wait_xe(q)
                    compute(q)
                    issue_scatter(q)
            return carry
        lax.fori_loop(0, NCH, chunk, 0)
        wait_scatter((NCH - 2) & 3)
        wait_scatter((NCH - 1) & 3)

        plsc.subcore_barrier()

        @pl.when(s < NRT)
        def _():
            pltpu.sync_copy(acc_sh.at[pl.ds(s * RP, RP)],
                            part_out.at[c, pl.ds(s * RP, RP)])
        if want_cnt:
            @pl.when(s < NRT)
            def _():
                pltpu.sync_copy(cnt_sh.at[pl.ds(s * RP, RP)],
                                zcnt.at[pl.ds(0, RP)])
                for r in range(8):
                    for g in range(LG):
                        zbuf[r, pl.ds(g * 16, 16)] = (
                            zcnt[pl.ds(r * 128 + g * 16, 16)])
                pltpu.sync_copy(zbuf, part_out.at[c, pl.ds(N + s * 8, 8)])

    return pl.kernel(body, out_type=part_ty, mesh=mesh,
                     scratch_types=scratch)


_edge_agg_cnt = _make_edge_agg(True)
_edge_agg = _make_edge_agg(False)

R = 1000


def _tc0_body(part, cnt0, cnt1, x, wl, bl, wr, out):
    summ = part[0] + part[1]
    deg = cnt0[...] + cnt1[...]
    agg = summ / jnp.maximum(deg, 1.0)
    h = (jnp.dot(agg, wl[...], preferred_element_type=jnp.float32) + bl[...]
         + jnp.dot(x[...], wr[...], preferred_element_type=jnp.float32))
    out[...] = jnp.maximum(h, 0.0)


_tc0 = pl.pallas_call(
    _tc0_body,
    grid=(N // R,),
    in_specs=[
        pl.BlockSpec((NC, R, D), lambda i: (0, i, 0)),
        pl.BlockSpec((R, 1), lambda i: (i, 0)),
        pl.BlockSpec((R, 1), lambda i: (i, 0)),
        pl.BlockSpec((R, D), lambda i: (i, 0)),
        pl.BlockSpec((D, D), lambda i: (0, 0)),
        pl.BlockSpec((1, D), lambda i: (0, 0)),
        pl.BlockSpec((D, D), lambda i: (0, 0)),
    ],
    out_specs=pl.BlockSpec((R, D), lambda i: (i, 0)),
    out_shape=jax.ShapeDtypeStruct((N, D), jnp.float32),
)


def _tc1_body(part, cnt0, cnt1, h, wl, bl, wr, wp, bp, out, g):
    i = pl.program_id(0)
    summ = part[0] + part[1]
    deg = cnt0[...] + cnt1[...]
    agg = summ / jnp.maximum(deg, 1.0)
    h2 = (jnp.dot(agg, wl[...], preferred_element_type=jnp.float32) + bl[...]
          + jnp.dot(h[...], wr[...], preferred_element_type=jnp.float32))

    @pl.when(i == 0)
    def _():
        g[...] = jnp.zeros_like(g)
    g[...] += jnp.sum(h2, axis=0, keepdims=True) * (1.0 / N)
    out[...] = (jnp.dot(jnp.maximum(h2, 0.0), wp[...],
                        preferred_element_type=jnp.float32) + bp[...])


_tc1 = pl.pallas_call(
    _tc1_body,
    grid=(N // R,),
    in_specs=[
        pl.BlockSpec((NC, R, D), lambda i: (0, i, 0)),
        pl.BlockSpec((R, 1), lambda i: (i, 0)),
        pl.BlockSpec((R, 1), lambda i: (i, 0)),
        pl.BlockSpec((R, D), lambda i: (i, 0)),
        pl.BlockSpec((D, D), lambda i: (0, 0)),
        pl.BlockSpec((1, D), lambda i: (0, 0)),
        pl.BlockSpec((D, D), lambda i: (0, 0)),
        pl.BlockSpec((D, C), lambda i: (0, 0)),
        pl.BlockSpec((1, C), lambda i: (0, 0)),
    ],
    out_specs=[
        pl.BlockSpec((R, C), lambda i: (i, 0)),
        pl.BlockSpec((1, D), lambda i: (0, 0)),
    ],
    out_shape=[
        jax.ShapeDtypeStruct((N, C), jnp.float32),
        jax.ShapeDtypeStruct((1, D), jnp.float32),
    ],
)


def kernel(x, edge_index, xe, W_l0, b_l0, W_r0, W_l1, b_l1, W_r1, W_proj,
           b_proj):
    src = edge_index[0]
    dst = edge_index[1]
    part0 = _edge_agg_cnt(x, src, dst, xe)
    cnt0 = part0[0, N:].reshape(NRT, 1024)[:, :RP].reshape(N, 1)
    cnt1 = part0[1, N:].reshape(NRT, 1024)[:, :RP].reshape(N, 1)
    h1 = _tc0(part0, cnt0, cnt1, x, W_l0, b_l0.reshape(1, D), W_r0)
    part1 = _edge_agg(h1, src, dst, xe)
    h_out, g = _tc1(part1, cnt0, cnt1, h1, W_l1, b_l1.reshape(1, D), W_r1,
                    W_proj, b_proj.reshape(1, C))
    return (h_out, g)


# speedup vs baseline: 3.9702x; 1.0000x over previous
import functools

import jax
import jax.numpy as jnp
from jax import lax
from jax.experimental import pallas as pl
from jax.experimental.pallas import tpu as pltpu
from jax.experimental.pallas import tpu_sc as plsc

N = 10000
E = 320000
D = 128
C = 64

NC = 2
NS = 16
NW = NC * NS
EP = E // NW
K = 40
NCH = EP // K
NQ = 4
RP = 1000
NRT = N // RP
RZ = 8
LG = D // 16


def _make_edge_agg(want_cnt):
    mesh = plsc.VectorSubcoreMesh(core_axis_name="c", subcore_axis_name="s")
    xrow_rows = N + 8 * NRT if want_cnt else N
    part_ty = jax.ShapeDtypeStruct((NC, xrow_rows, D), jnp.float32)
    scratch = [
        pltpu.VMEM_SHARED((N, D), jnp.float32),
        pltpu.VMEM((NQ, K), jnp.int32),
        pltpu.VMEM((NQ, K), jnp.int32),
        pltpu.VMEM((NQ, K, D), jnp.float32),
        pltpu.VMEM((NQ, K, D), jnp.float32),
        pltpu.VMEM((RZ, D), jnp.float32),
        pltpu.SemaphoreType.DMA((NQ,)),
        pltpu.SemaphoreType.DMA((NQ,)),
        pltpu.SemaphoreType.DMA((NQ,)),
        pltpu.SemaphoreType.DMA((NQ,)),
        pltpu.SemaphoreType.DMA((NQ,)),
    ]
    if want_cnt:
        scratch += [
            pltpu.VMEM_SHARED((N,), jnp.float32),
            pltpu.VMEM((48,), jnp.float32),
            pltpu.VMEM((1024,), jnp.float32),
            pltpu.SemaphoreType.DMA((NQ,)),
        ]

    def body(x_hbm, src_hbm, dst_hbm, xe_hbm, part_out, *rest):
        if want_cnt:
            (acc_sh, idx_s, idx_d, xrows, xe_v, zbuf,
             isem, dsem, gsem, xesem, ssem, cnt_sh, ones_v, zcnt,
             csem) = rest
        else:
            (acc_sh, idx_s, idx_d, xrows, xe_v, zbuf,
             isem, dsem, gsem, xesem, ssem) = rest
        c = lax.axis_index("c")
        s = lax.axis_index("s")
        wid = s * NC + c
        zero16 = jnp.zeros((16,), jnp.float32)

        def zrow(r, carry):
            for g in range(LG):
                zbuf[r, pl.ds(g * 16, 16)] = zero16
            return carry
        lax.fori_loop(0, RZ, zrow, 0)

        @pl.when(s < NRT)
        def _():
            def zacc(j, carry):
                pltpu.sync_copy(zbuf, acc_sh.at[pl.ds(s * RP + j * RZ, RZ)])
                return carry
            lax.fori_loop(0, RP // RZ, zacc, 0)

        if want_cnt:
            def zc(i, carry):
                zcnt[pl.ds(i * 16, 16)] = zero16
                return carry
            lax.fori_loop(0, 1024 // 16, zc, 0)

            def fones(i, carry):
                ones_v[pl.ds(i * 16, 16)] = zero16 + 1.0
                return carry
            lax.fori_loop(0, 48 // 16, fones, 0)

            @pl.when(s < NRT)
            def _():
                pltpu.sync_copy(zcnt.at[pl.ds(0, RP)],
                                cnt_sh.at[pl.ds(s * RP, RP)])

        plsc.subcore_barrier()

        ebase = wid * EP

        def issue_idx_s(j, q):
            pltpu.async_copy(src_hbm.at[pl.ds(ebase + j * K, K)],
                             idx_s.at[q], isem.at[q])

        def issue_idx_d(j, q):
            pltpu.async_copy(dst_hbm.at[pl.ds(ebase + j * K, K)],
                             idx_d.at[q], dsem.at[q])

        def issue_xe(j, q):
            pltpu.async_copy(xe_hbm.at[pl.ds(ebase + j * K, K)],
                             xe_v.at[q], xesem.at[q])

        def wait_xe(q):
            pltpu.make_async_copy(xe_hbm.at[pl.ds(0, K)], xe_v.at[q],
                                  xesem.at[q]).wait()

        def issue_gather(q):
            pltpu.make_async_copy(src_hbm.at[pl.ds(0, K)], idx_s.at[q],
                                  isem.at[q]).wait()
            pltpu.async_copy(x_hbm.at[idx_s.at[q]], xrows.at[q],
                             gsem.at[q])

        def wait_gather(q):
            pltpu.make_async_copy(x_hbm.at[idx_s.at[q]], xrows.at[q],
                                  gsem.at[q]).wait()

        def issue_scatter(q):
            pltpu.make_async_copy(dst_hbm.at[pl.ds(0, K)], idx_d.at[q],
                                  dsem.at[q]).wait()
            pltpu.async_copy(xrows.at[q], acc_sh.at[idx_d.at[q]],
                             ssem.at[q], add=True)
            if want_cnt:
                pltpu.async_copy(ones_v.at[pl.ds(0, K)],
                                 cnt_sh.at[idx_d.at[q]],
                                 csem.at[q], add=True)

        def wait_scatter(q):
            pltpu.make_async_copy(xrows.at[q], acc_sh.at[idx_d.at[q]],
                                  ssem.at[q]).wait()
            if want_cnt:
                pltpu.make_async_copy(ones_v.at[pl.ds(0, K)],
                                      cnt_sh.at[idx_d.at[q]],
                                      csem.at[q]).wait()

        def compute(q):
            def row(r, rc):
                for g in range(LG):
                    sl = pl.ds(g * 16, 16)
                    xrows[q, r, sl] = jnp.maximum(
                        xrows[q, r, sl] + xe_v[q, r, sl], 0.0)
                return rc
            lax.fori_loop(0, K, row, 0, unroll=2)

        issue_idx_s(0, 0)
        issue_idx_s(1, 1)
        issue_idx_s(2, 2)
        issue_idx_d(0, 0)
        issue_idx_d(1, 1)
        issue_xe(0, 0)
        issue_xe(1, 1)
        issue_gather(0)
        issue_gather(1)

        def chunk(j, carry):
            u = j & 3
            for q in range(NQ):
                @pl.when(u == q)
                def _(q=q):
                    @pl.when(j >= 2)
                    def _():
                        wait_scatter((q + 2) & 3)

                    @pl.when(j + 3 < NCH)
                    def _():
                        issue_idx_s(j + 3, (q + 3) & 3)

                    @pl.when(j + 2 < NCH)
                    def _():
                        issue_idx_d(j + 2, (q + 2) & 3)
                        issue_xe(j + 2, (q + 2) & 3)
                        issue_gather((q + 2) & 3)

                    wait_gather(q)
                    wait_xe(q)
                    compute(q)
                    issue_scatter(q)
            return carry
        lax.fori_loop(0, NCH, chunk, 0)
        wait_scatter((NCH - 2) & 3)
        wait_scatter((NCH - 1) & 3)

        plsc.subcore_barrier()

        @pl.when(s < NRT)
        def _():
            pltpu.sync_copy(acc_sh.at[pl.ds(s * RP, RP)],
                            part_out.at[c, pl.ds(s * RP, RP)])
        if want_cnt:
            @pl.when(s < NRT)
            def _():
                pltpu.sync_copy(cnt_sh.at[pl.ds(s * RP, RP)],
                                zcnt.at[pl.ds(0, RP)])
                for r in range(8):
                    for g in range(LG):
                        zbuf[r, pl.ds(g * 16, 16)] = (
                            zcnt[pl.ds(r * 128 + g * 16, 16)])
                pltpu.sync_copy(zbuf, part_out.at[c, pl.ds(N + s * 8, 8)])

    return pl.kernel(body, out_type=part_ty, mesh=mesh,
                     scratch_types=scratch)


_edge_agg_cnt = _make_edge_agg(True)
_edge_agg = _make_edge_agg(False)

R = 1000


def _tc0_body(part, cnt0, cnt1, x, wl, bl, wr, out):
    summ = part[0] + part[1]
    deg = cnt0[...] + cnt1[...]
    agg = summ / jnp.maximum(deg, 1.0)
    h = (jnp.dot(agg, wl[...], preferred_element_type=jnp.float32) + bl[...]
         + jnp.dot(x[...], wr[...], preferred_element_type=jnp.float32))
    out[...] = jnp.maximum(h, 0.0)


_tc0 = pl.pallas_call(
    _tc0_body,
    grid=(N // R,),
    in_specs=[
        pl.BlockSpec((NC, R, D), lambda i: (0, i, 0)),
        pl.BlockSpec((R, 1), lambda i: (i, 0)),
        pl.BlockSpec((R, 1), lambda i: (i, 0)),
        pl.BlockSpec((R, D), lambda i: (i, 0)),
        pl.BlockSpec((D, D), lambda i: (0, 0)),
        pl.BlockSpec((1, D), lambda i: (0, 0)),
        pl.BlockSpec((D, D), lambda i: (0, 0)),
    ],
    out_specs=pl.BlockSpec((R, D), lambda i: (i, 0)),
    out_shape=jax.ShapeDtypeStruct((N, D), jnp.float32),
)


def _tc1_body(part, cnt0, cnt1, h, wl, bl, wr, wp, bp, out, g):
    i = pl.program_id(0)
    summ = part[0] + part[1]
    deg = cnt0[...] + cnt1[...]
    agg = summ / jnp.maximum(deg, 1.0)
    h2 = (jnp.dot(agg, wl[...], preferred_element_type=jnp.float32) + bl[...]
          + jnp.dot(h[...], wr[...], preferred_element_type=jnp.float32))

    @pl.when(i == 0)
    def _():
        g[...] = jnp.zeros_like(g)
    g[...] += jnp.sum(h2, axis=0, keepdims=True) * (1.0 / N)
    out[...] = (jnp.dot(jnp.maximum(h2, 0.0), wp[...],
                        preferred_element_type=jnp.float32) + bp[...])


_tc1 = pl.pallas_call(
    _tc1_body,
    grid=(N // R,),
    in_specs=[
        pl.BlockSpec((NC, R, D), lambda i: (0, i, 0)),
        pl.BlockSpec((R, 1), lambda i: (i, 0)),
        pl.BlockSpec((R, 1), lambda i: (i, 0)),
        pl.BlockSpec((R, D), lambda i: (i, 0)),
        pl.BlockSpec((D, D), lambda i: (0, 0)),
        pl.BlockSpec((1, D), lambda i: (0, 0)),
        pl.BlockSpec((D, D), lambda i: (0, 0)),
        pl.BlockSpec((D, C), lambda i: (0, 0)),
        pl.BlockSpec((1, C), lambda i: (0, 0)),
    ],
    out_specs=[
        pl.BlockSpec((R, C), lambda i: (i, 0)),
        pl.BlockSpec((1, D), lambda i: (0, 0)),
    ],
    out_shape=[
        jax.ShapeDtypeStruct((N, C), jnp.float32),
        jax.ShapeDtypeStruct((1, D), jnp.float32),
    ],
)


def kernel(x, edge_index, xe, W_l0, b_l0, W_r0, W_l1, b_l1, W_r1, W_proj,
           b_proj):
    src = edge_index[0]
    dst = edge_index[1]
    part0 = _edge_agg_cnt(x, src, dst, xe)
    cnt0 = part0[0, N:].reshape(NRT, 1024)[:, :RP].reshape(N, 1)
    cnt1 = part0[1, N:].reshape(NRT, 1024)[:, :RP].reshape(N, 1)
    h1 = _tc0(part0, cnt0, cnt1, x, W_l0, b_l0.reshape(1, D), W_r0)
    part1 = _edge_agg(h1, src, dst, xe)
    h_out, g = _tc1(part1, cnt0, cnt1, h1, W_l1, b_l1.reshape(1, D), W_r1,
                    W_proj, b_proj.reshape(1, C))
    return (h_out, g)
